# baseline (speedup 1.0000x reference)
import functools

import jax
import jax.numpy as jnp
from jax import lax
from jax.experimental import pallas as pl
from jax.experimental.pallas import tpu as pltpu
from jax.experimental.pallas import tpu_sc as plsc

VOCAB = 1000
B, T = 1024, 50
NVR = VOCAB // 8
NBC = B // 128

_info = plsc.get_sparse_core_info()
NC, NS = _info.num_cores, _info.num_subcores
NW = NC * NS
NQ = -(-NVR // NW)

_mesh = plsc.VectorSubcoreMesh(core_axis_name="c", subcore_axis_name="s")


@functools.partial(
    pl.kernel,
    mesh=_mesh,
    out_type=jax.ShapeDtypeStruct((T, NVR, NBC, 8, 128), jnp.float32),
    scratch_types=[
        pltpu.VMEM((T, B), jnp.int32),
        pltpu.VMEM((8 * VOCAB,), jnp.float32),
        pltpu.VMEM((8 * VOCAB,), jnp.float32),
        pltpu.VMEM((NBC, 8, 128), jnp.float32),
        pltpu.VMEM((NBC, 8, 128), jnp.float32),
        pltpu.SemaphoreType.DMA,
        pltpu.SemaphoreType.DMA,
        pltpu.SemaphoreType.DMA,
        pltpu.SemaphoreType.DMA,
        pltpu.SemaphoreType.DMA,
    ],
    compiler_params=pltpu.CompilerParams(
        use_tc_tiling_on_sc=False, needs_layout_passes=False
    ),
)
def _tgather(
    tabT_hbm, idxT_hbm, out_hbm,
    idx_v, slab_a, slab_b, ob0, ob1, sem0, sem1, isem, ssem_a, ssem_b,
):
    w = lax.axis_index("s") * NC + lax.axis_index("c")
    slabs = (slab_a, slab_b)
    ssems = (ssem_a, ssem_b)
    pltpu.async_copy(idxT_hbm, idx_v, isem)
    pltpu.async_copy(tabT_hbm.at[pl.ds(w * 8 * VOCAB, 8 * VOCAB)], slab_a, ssem_a)
    pltpu.make_async_copy(idxT_hbm, idx_v, isem).wait()

    def compute(t, vr, buf, sem):
        def bcbody(bc, c):
            base = bc * 128
            rows = [idx_v[t, pl.ds(base + g * 16, 16)] for g in range(8)]
            DEPTH = 12
            pend = []
            for g in range(8):
                for vi in range(8):
                    vals = plsc.load_gather(slab_v, [rows[g] + vi * VOCAB])
                    pend.append((vals, g, vi))
                    if len(pend) > DEPTH:
                        v_, g_, vi_ = pend.pop(0)
                        buf[bc, vi_, pl.ds(g_ * 16, 16)] = v_
            for v_, g_, vi_ in pend:
                buf[bc, vi_, pl.ds(g_ * 16, 16)] = v_
            return c

        lax.fori_loop(0, NBC, bcbody, 0)
        pltpu.async_copy(buf, out_hbm.at[t, vr], sem)

    def drain(buf, sem):
        pltpu.make_async_copy(out_hbm.at[0, 0], buf, sem).wait()

    for q in range(NQ):
        vr = w + NW * q
        slab_v = slabs[q % 2]
        ssem = ssems[q % 2]

        @pl.when(vr < NVR)
        def _round(q=q, vr=vr, slab_v=slab_v, ssem=ssem):
            pltpu.make_async_copy(
                tabT_hbm.at[pl.ds(0, 8 * VOCAB)], slab_v, ssem
            ).wait()
            if q + 1 < NQ:
                vr_n = w + NW * (q + 1)

                def _prefetch():
                    pltpu.async_copy(
                        tabT_hbm.at[pl.ds(vr_n * 8 * VOCAB, 8 * VOCAB)],
                        slabs[(q + 1) % 2],
                        ssems[(q + 1) % 2],
                    )

                pl.when(vr_n < NVR)(_prefetch)

            def pbody(p, carry):
                t = 2 * p
                if q == 0:
                    pl.when(p > 0)(lambda: drain(ob0, sem0))
                else:
                    drain(ob0, sem0)
                compute(t, vr, ob0, sem0)
                if q == 0:
                    pl.when(p > 0)(lambda: drain(ob1, sem1))
                else:
                    drain(ob1, sem1)
                compute(t + 1, vr, ob1, sem1)
                return carry

            lax.fori_loop(0, T // 2, pbody, 0)

    drain(ob0, sem0)
    drain(ob1, sem1)


def kernel(idx, token_embedding_table):
    idxT = jnp.transpose(idx.astype(jnp.int32))
    tabT = jnp.transpose(token_embedding_table).reshape(-1)
    out5 = _tgather(tabT, idxT)
    z = jnp.transpose(out5, (2, 4, 0, 1, 3))
    return z.reshape(B, T, VOCAB)


# speedup vs baseline: 1.0423x; 1.0423x over previous
import functools

import jax
import jax.numpy as jnp
from jax import lax
from jax.experimental import pallas as pl
from jax.experimental.pallas import tpu as pltpu
from jax.experimental.pallas import tpu_sc as plsc

VOCAB = 1000
B, T = 1024, 50
NVR = VOCAB // 8
NBC = B // 128

_info = plsc.get_sparse_core_info()
NC, NS = _info.num_cores, _info.num_subcores
NW = NC * NS
NQ = -(-NVR // NW)

_mesh = plsc.VectorSubcoreMesh(core_axis_name="c", subcore_axis_name="s")


@functools.partial(
    pl.kernel,
    mesh=_mesh,
    out_type=jax.ShapeDtypeStruct((T, NVR, NBC, 8, 128), jnp.float32),
    scratch_types=[
        pltpu.VMEM((T, B), jnp.int32),
        pltpu.VMEM((8 * VOCAB,), jnp.float32),
        pltpu.VMEM((8 * VOCAB,), jnp.float32),
        pltpu.VMEM((NBC, 8, 128), jnp.float32),
        pltpu.VMEM((NBC, 8, 128), jnp.float32),
        pltpu.SemaphoreType.DMA,
        pltpu.SemaphoreType.DMA,
        pltpu.SemaphoreType.DMA,
        pltpu.SemaphoreType.DMA,
        pltpu.SemaphoreType.DMA,
    ],
    compiler_params=pltpu.CompilerParams(
        use_tc_tiling_on_sc=False, needs_layout_passes=False
    ),
)
def _tgather(
    tabT_hbm, idxT_hbm, out_hbm,
    idx_v, slab_a, slab_b, ob0, ob1, sem0, sem1, isem, ssem_a, ssem_b,
):
    w = lax.axis_index("s") * NC + lax.axis_index("c")
    slabs = (slab_a, slab_b)
    ssems = (ssem_a, ssem_b)
    pltpu.async_copy(idxT_hbm, idx_v, isem)
    pltpu.async_copy(tabT_hbm.at[pl.ds(w * 8 * VOCAB, 8 * VOCAB)], slab_a, ssem_a)
    pltpu.make_async_copy(idxT_hbm, idx_v, isem).wait()

    def compute(t, vr, buf, sem):
        def bcbody(bc, c):
            base = bc * 128
            rows = [idx_v[t, pl.ds(base + g * 16, 16)] for g in range(8)]
            DEPTH = 6
            pend = []
            for g in range(8):
                for vi in range(8):
                    vals = plsc.load_gather(slab_v, [rows[g] + vi * VOCAB])
                    pend.append((vals, g, vi))
                    if len(pend) > DEPTH:
                        v_, g_, vi_ = pend.pop(0)
                        buf[bc, vi_, pl.ds(g_ * 16, 16)] = v_
            for v_, g_, vi_ in pend:
                buf[bc, vi_, pl.ds(g_ * 16, 16)] = v_
            return c

        lax.fori_loop(0, NBC, bcbody, 0)
        pltpu.async_copy(buf, out_hbm.at[t, vr], sem)

    def drain(buf, sem):
        pltpu.make_async_copy(out_hbm.at[0, 0], buf, sem).wait()

    for q in range(NQ):
        vr = w + NW * q
        slab_v = slabs[q % 2]
        ssem = ssems[q % 2]

        @pl.when(vr < NVR)
        def _round(q=q, vr=vr, slab_v=slab_v, ssem=ssem):
            pltpu.make_async_copy(
                tabT_hbm.at[pl.ds(0, 8 * VOCAB)], slab_v, ssem
            ).wait()
            if q + 1 < NQ:
                vr_n = w + NW * (q + 1)

                def _prefetch():
                    pltpu.async_copy(
                        tabT_hbm.at[pl.ds(vr_n * 8 * VOCAB, 8 * VOCAB)],
                        slabs[(q + 1) % 2],
                        ssems[(q + 1) % 2],
                    )

                pl.when(vr_n < NVR)(_prefetch)

            def pbody(p, carry):
                t = 2 * p
                if q == 0:
                    pl.when(p > 0)(lambda: drain(ob0, sem0))
                else:
                    drain(ob0, sem0)
                compute(t, vr, ob0, sem0)
                if q == 0:
                    pl.when(p > 0)(lambda: drain(ob1, sem1))
                else:
                    drain(ob1, sem1)
                compute(t + 1, vr, ob1, sem1)
                return carry

            lax.fori_loop(0, T // 2, pbody, 0)

    drain(ob0, sem0)
    drain(ob1, sem1)


def kernel(idx, token_embedding_table):
    idxT = jnp.transpose(idx.astype(jnp.int32))
    tabT = jnp.transpose(token_embedding_table).reshape(-1)
    out5 = _tgather(tabT, idxT)
    z = jnp.transpose(out5, (2, 4, 0, 1, 3))
    return z.reshape(B, T, VOCAB)
